# baseline (speedup 1.0000x reference)
import functools

import jax
import jax.numpy as jnp
from jax import lax
from jax.experimental import pallas as pl
from jax.experimental.pallas import tpu as pltpu
from jax.experimental.pallas import tpu_sc as plsc

VOCAB = 1000000
MAXLEN = 200
EMBED = 64
BATCH = 1024

_info = plsc.get_sparse_core_info()
_NC, _NS, _L = _info.num_cores, _info.num_subcores, _info.num_lanes
_NW = _NC * _NS
_ROWS_PER_W = BATCH // _NW


def _build(B, L, E):
    assert B % _NW == 0 and E % _L == 0
    rows_per_w = B // _NW
    mesh = plsc.VectorSubcoreMesh(core_axis_name="c", subcore_axis_name="s")

    @functools.partial(
        pl.kernel,
        mesh=mesh,
        compiler_params=pltpu.CompilerParams(use_tc_tiling_on_sc=False),
        out_type=jax.ShapeDtypeStruct((B, L, E), jnp.float32),
        scratch_types=[
            pltpu.VMEM((L,), jnp.int32),
            pltpu.VMEM((L, E), jnp.float32),
            pltpu.VMEM((L, E), jnp.float32),
            pltpu.SemaphoreType.DMA,
        ],
    )
    def k(x_hbm, tok_hbm, pos_hbm, out_hbm, idx_v, rows_v, pos_v, sem):
        wid = lax.axis_index("s") * _NC + lax.axis_index("c")
        base = wid * rows_per_w
        pltpu.sync_copy(pos_hbm, pos_v)

        def per_row(r, carry):
            row = base + r
            pltpu.sync_copy(x_hbm.at[row], idx_v)
            cp1 = pltpu.async_copy(
                tok_hbm.at[idx_v.at[pl.ds(0, 128)]], rows_v.at[pl.ds(0, 128)], sem)
            cp2 = pltpu.async_copy(
                tok_hbm.at[idx_v.at[pl.ds(128, L - 128)]],
                rows_v.at[pl.ds(128, L - 128)], sem)
            cp1.wait()
            cp2.wait()

            def add_pos(l, c):
                for j in range(E // _L):
                    sl = pl.ds(j * _L, _L)
                    rows_v[l, sl] = rows_v[l, sl] + pos_v[l, sl]
                return c

            lax.fori_loop(0, L, add_pos, 0)
            pltpu.sync_copy(rows_v, out_hbm.at[row])
            return carry

        lax.fori_loop(0, rows_per_w, per_row, 0)

    return k


_emb = _build(BATCH, MAXLEN, EMBED)


def kernel(x, token_table, pos_table):
    return _emb(x.astype(jnp.int32), token_table, pos_table)


# speedup vs baseline: 1.0571x; 1.0571x over previous
import functools

import jax
import jax.numpy as jnp
from jax import lax
from jax.experimental import pallas as pl
from jax.experimental.pallas import tpu as pltpu
from jax.experimental.pallas import tpu_sc as plsc

VOCAB = 1000000
MAXLEN = 200
EMBED = 64
BATCH = 1024

_info = plsc.get_sparse_core_info()
_NC, _NS, _L = _info.num_cores, _info.num_subcores, _info.num_lanes
_NW = _NC * _NS


def _build(B, L, E):
    assert B % _NW == 0 and E % _L == 0
    rows_per_w = B // _NW
    assert rows_per_w % 2 == 0
    mesh = plsc.VectorSubcoreMesh(core_axis_name="c", subcore_axis_name="s")

    @functools.partial(
        pl.kernel,
        mesh=mesh,
        compiler_params=pltpu.CompilerParams(use_tc_tiling_on_sc=False),
        out_type=jax.ShapeDtypeStruct((B, L, E), jnp.float32),
        scratch_types=[
            pltpu.VMEM((rows_per_w, L), jnp.int32),
            pltpu.VMEM((L, E), jnp.float32),
            pltpu.VMEM((L, E), jnp.float32),
            pltpu.VMEM((L, E), jnp.float32),
            pltpu.SemaphoreType.DMA,
            pltpu.SemaphoreType.DMA,
            pltpu.SemaphoreType.DMA,
            pltpu.SemaphoreType.DMA,
        ],
    )
    def k(x_hbm, tok_hbm, pos_hbm, out_hbm, idx_all, rows0, rows1, pos_v,
          gsem0, gsem1, osem0, osem1):
        wid = lax.axis_index("s") * _NC + lax.axis_index("c")
        base = wid * rows_per_w
        bufs = (rows0, rows1)
        gsems = (gsem0, gsem1)
        osems = (osem0, osem1)

        pltpu.sync_copy(x_hbm.at[pl.ds(base, rows_per_w)], idx_all)
        pltpu.sync_copy(pos_hbm, pos_v)

        def fire_gather(r, b):
            pltpu.async_copy(
                tok_hbm.at[idx_all.at[r, pl.ds(0, 128)]],
                bufs[b].at[pl.ds(0, 128)], gsems[b])
            pltpu.async_copy(
                tok_hbm.at[idx_all.at[r, pl.ds(128, L - 128)]],
                bufs[b].at[pl.ds(128, L - 128)], gsems[b])

        def wait_gather(b):
            pltpu.make_async_copy(
                tok_hbm.at[idx_all.at[0, pl.ds(0, 128)]],
                bufs[b].at[pl.ds(0, 128)], gsems[b]).wait()
            pltpu.make_async_copy(
                tok_hbm.at[idx_all.at[0, pl.ds(128, L - 128)]],
                bufs[b].at[pl.ds(128, L - 128)], gsems[b]).wait()

        def wait_out(b):
            pltpu.make_async_copy(bufs[b], out_hbm.at[0], osems[b]).wait()

        fire_gather(0, 0)
        fire_gather(1, 1)

        @pl.loop(0, rows_per_w, step=2)
        def per_pair(g):
            for b in range(2):
                r = g + b
                wait_gather(b)

                @plsc.parallel_loop(0, L, unroll=2)
                def add_pos(l):
                    for j in range(E // _L):
                        sl = pl.ds(j * _L, _L)
                        bufs[b][l, sl] = bufs[b][l, sl] + pos_v[l, sl]

                pltpu.async_copy(bufs[b], out_hbm.at[base + r], osems[b])

            @pl.when(g + 2 < rows_per_w)
            def _():
                for b in range(2):
                    wait_out(b)
                    fire_gather(g + 2 + b, b)

        wait_out(0)
        wait_out(1)

    return k


_emb = _build(BATCH, MAXLEN, EMBED)


def kernel(x, token_table, pos_table):
    return _emb(x.astype(jnp.int32), token_table, pos_table)


# speedup vs baseline: 1.2388x; 1.1718x over previous
import functools

import jax
import jax.numpy as jnp
from jax import lax
from jax.experimental import pallas as pl
from jax.experimental.pallas import tpu as pltpu
from jax.experimental.pallas import tpu_sc as plsc

VOCAB = 1000000
MAXLEN = 200
EMBED = 64
BATCH = 1024
EPAD = 128

_info = plsc.get_sparse_core_info()
_NC, _NS, _L = _info.num_cores, _info.num_subcores, _info.num_lanes
_NW = _NC * _NS


def _build(B, L, E):
    assert B % _NW == 0 and E % _L == 0
    rows_per_w = B // _NW
    assert rows_per_w % 2 == 0
    mesh = plsc.VectorSubcoreMesh(core_axis_name="c", subcore_axis_name="s")

    @functools.partial(
        pl.kernel,
        mesh=mesh,
        compiler_params=pltpu.CompilerParams(use_tc_tiling_on_sc=True),
        out_type=jax.ShapeDtypeStruct((B, L, E), jnp.float32),
        scratch_types=[
            pltpu.VMEM((rows_per_w * L,), jnp.int32),
            pltpu.VMEM((L, E), jnp.float32),
            pltpu.VMEM((L, E), jnp.float32),
            pltpu.VMEM((L, E), jnp.float32),
            pltpu.SemaphoreType.DMA,
            pltpu.SemaphoreType.DMA,
            pltpu.SemaphoreType.DMA,
            pltpu.SemaphoreType.DMA,
        ],
    )
    def k(x_hbm, tok_hbm, pos_hbm, out_hbm, idx_all, rows0, rows1, pos_v,
          gsem0, gsem1, osem0, osem1):
        wid = lax.axis_index("s") * _NC + lax.axis_index("c")
        base = wid * rows_per_w
        bufs = (rows0, rows1)
        gsems = (gsem0, gsem1)
        osems = (osem0, osem1)

        pltpu.sync_copy(x_hbm.at[pl.ds(base * L, rows_per_w * L)], idx_all)
        pltpu.sync_copy(pos_hbm, pos_v)

        def fire_gather(r, b):
            pltpu.async_copy(
                tok_hbm.at[idx_all.at[pl.ds(r * L, 128)]],
                bufs[b].at[pl.ds(0, 128)], gsems[b])
            pltpu.async_copy(
                tok_hbm.at[idx_all.at[pl.ds(r * L + 128, L - 128)]],
                bufs[b].at[pl.ds(128, L - 128)], gsems[b])

        def wait_gather(b):
            pltpu.make_async_copy(
                tok_hbm.at[idx_all.at[pl.ds(0, 128)]],
                bufs[b].at[pl.ds(0, 128)], gsems[b]).wait()
            pltpu.make_async_copy(
                tok_hbm.at[idx_all.at[pl.ds(0, L - 128)]],
                bufs[b].at[pl.ds(128, L - 128)], gsems[b]).wait()

        def wait_out(b):
            pltpu.make_async_copy(bufs[b], out_hbm.at[0], osems[b]).wait()

        fire_gather(0, 0)
        fire_gather(1, 1)

        @pl.loop(0, rows_per_w, step=2)
        def per_pair(g):
            for b in range(2):
                r = g + b
                wait_gather(b)

                @plsc.parallel_loop(0, L, unroll=2)
                def add_pos(l):
                    for j in range(EMBED // _L):
                        sl = pl.ds(j * _L, _L)
                        bufs[b][l, sl] = bufs[b][l, sl] + pos_v[l, sl]

                pltpu.async_copy(bufs[b], out_hbm.at[base + r], osems[b])

            @pl.when(g + 2 < rows_per_w)
            def _():
                for b in range(2):
                    wait_out(b)
                    fire_gather(g + 2 + b, b)

        wait_out(0)
        wait_out(1)

    return k


_emb = _build(BATCH, MAXLEN, EPAD)


def kernel(x, token_table, pos_table):
    xf = x.reshape(-1).astype(jnp.int32)
    tt = jnp.pad(token_table, ((0, 0), (0, EPAD - EMBED)))
    pp = jnp.pad(pos_table, ((0, 0), (0, EPAD - EMBED)))
    out = _emb(xf, tt, pp)
    return out[:, :, :EMBED]


# speedup vs baseline: 1.3745x; 1.1096x over previous
import functools

import jax
import jax.numpy as jnp
from jax import lax
from jax.experimental import pallas as pl
from jax.experimental.pallas import tpu as pltpu
from jax.experimental.pallas import tpu_sc as plsc

VOCAB = 1000000
MAXLEN = 200
EMBED = 64
BATCH = 1024
EPAD = 128

_info = plsc.get_sparse_core_info()
_NC, _NS, _L = _info.num_cores, _info.num_subcores, _info.num_lanes
_NW = _NC * _NS


def _build(B, L, E):
    assert B % _NW == 0 and E % _L == 0
    rows_per_w = B // _NW
    assert rows_per_w % 2 == 0
    mesh = plsc.VectorSubcoreMesh(core_axis_name="c", subcore_axis_name="s")

    @functools.partial(
        pl.kernel,
        mesh=mesh,
        compiler_params=pltpu.CompilerParams(use_tc_tiling_on_sc=True),
        out_type=jax.ShapeDtypeStruct((B, L, E), jnp.float32),
        scratch_types=[
            pltpu.VMEM((rows_per_w * L,), jnp.int32),
            pltpu.VMEM((L, E), jnp.float32),
            pltpu.VMEM((L, E), jnp.float32),
            pltpu.VMEM((L, E), jnp.float32),
            pltpu.SemaphoreType.DMA,
            pltpu.SemaphoreType.DMA,
            pltpu.SemaphoreType.DMA,
            pltpu.SemaphoreType.DMA,
        ],
    )
    def k(x_hbm, tok_hbm, pos_hbm, out_hbm, idx_all, rows0, rows1, pos_v,
          gsem0, gsem1, osem0, osem1):
        wid = lax.axis_index("s") * _NC + lax.axis_index("c")
        base = wid * rows_per_w
        bufs = (rows0, rows1)
        gsems = (gsem0, gsem1)
        osems = (osem0, osem1)

        pltpu.sync_copy(x_hbm.at[pl.ds(base * L, rows_per_w * L)], idx_all)
        pltpu.sync_copy(pos_hbm, pos_v)

        def fire_gather(r, b):
            pltpu.async_copy(
                tok_hbm.at[idx_all.at[pl.ds(r * L, 128)]],
                bufs[b].at[pl.ds(0, 128)], gsems[b])
            pltpu.async_copy(
                tok_hbm.at[idx_all.at[pl.ds(r * L + 128, L - 128)]],
                bufs[b].at[pl.ds(128, L - 128)], gsems[b])

        def wait_gather(b):
            pltpu.make_async_copy(
                tok_hbm.at[idx_all.at[pl.ds(0, 128)]],
                bufs[b].at[pl.ds(0, 128)], gsems[b]).wait()
            pltpu.make_async_copy(
                tok_hbm.at[idx_all.at[pl.ds(0, L - 128)]],
                bufs[b].at[pl.ds(128, L - 128)], gsems[b]).wait()

        def wait_out(b):
            pltpu.make_async_copy(bufs[b], out_hbm.at[0], osems[b]).wait()

        fire_gather(0, 0)
        fire_gather(1, 1)

        @pl.loop(0, rows_per_w, step=2)
        def per_pair(g):
            for b in range(2):
                r = g + b
                wait_gather(b)

                @plsc.parallel_loop(0, L, unroll=2)
                def add_pos(l):
                    for j in range(EMBED // _L):
                        sl = pl.ds(j * _L, _L)
                        bufs[b][l, sl] = bufs[b][l, sl] + pos_v[l, sl]

                pltpu.async_copy(bufs[b], out_hbm.at[base + r], osems[b])

            @pl.when(g + 2 < rows_per_w)
            def _():
                for b in range(2):
                    wait_out(b)
                    fire_gather(g + 2 + b, b)

        wait_out(0)
        wait_out(1)

    return k


_emb = _build(BATCH, MAXLEN, EPAD)

_TP_VB = 2048


def _tp_body(x_ref, o_ref):
    xt = jnp.transpose(x_ref[...], (1, 0))
    o_ref[...] = jnp.concatenate(
        [xt, jnp.zeros((xt.shape[0], EPAD - EMBED), jnp.float32)], axis=1)


_transpose_pad = pl.pallas_call(
    _tp_body,
    grid=(pl.cdiv(VOCAB, _TP_VB),),
    in_specs=[pl.BlockSpec((EMBED, _TP_VB), lambda i: (0, i))],
    out_specs=pl.BlockSpec((_TP_VB, EPAD), lambda i: (i, 0)),
    out_shape=jax.ShapeDtypeStruct((VOCAB, EPAD), jnp.float32),
)


def kernel(x, token_table, pos_table):
    xf = x.reshape(-1).astype(jnp.int32)
    tt = _transpose_pad(token_table.T)
    pp = jnp.pad(pos_table, ((0, 0), (0, EPAD - EMBED)))
    out = _emb(xf, tt, pp)
    return out[:, :, :EMBED]
